# baseline (speedup 1.0000x reference)
import functools

import jax
import jax.numpy as jnp
from jax import lax
from jax.experimental import pallas as pl
from jax.experimental.pallas import tpu as pltpu
from jax.experimental.pallas import tpu_sc as plsc

NC, NS = 2, 16
NW = NC * NS
D = 64
SEQ = 50
B = 16384
BPW = B // NW
NBLK = BPW // 128


@jax.jit
def _gather(idx2, table2):
    mesh = plsc.VectorSubcoreMesh(
        core_axis_name="c", subcore_axis_name="s", num_cores=NC, num_subcores=NS
    )

    @functools.partial(
        pl.kernel,
        mesh=mesh,
        out_type=jax.ShapeDtypeStruct((SEQ, D, B // 128, 128), jnp.float32),
        scratch_types=[
            pltpu.VMEM((SEQ, BPW), jnp.int32),
            pltpu.VMEM((2, 128, D), jnp.float32),
            pltpu.VMEM((2, D, 128), jnp.float32),
            pltpu.SemaphoreType.DMA,
            pltpu.SemaphoreType.DMA,
            pltpu.SemaphoreType.DMA,
            pltpu.SemaphoreType.DMA,
        ],
        compiler_params=pltpu.CompilerParams(
            use_tc_tiling_on_sc=False, needs_layout_passes=False
        ),
    )
    def kern(idx_hbm, table_hbm, out_hbm, idx_v, rows_v, rowsT_v, g0, g1, o0, o1):
        gsem = (g0, g1)
        osem = (o0, o1)
        wid = lax.axis_index("s") * NC + lax.axis_index("c")
        pltpu.sync_copy(idx_hbm.at[:, pl.ds(wid * BPW, BPW)], idx_v)

        def gather_desc(w, blk, cur):
            return pltpu.make_async_copy(
                table_hbm.at[idx_v.at[w, pl.ds(blk * 128, 128)]],
                rows_v.at[cur],
                gsem[cur],
            )

        def out_desc(w, blk, cur):
            return pltpu.make_async_copy(
                rowsT_v.at[cur],
                out_hbm.at[w, pl.ds(0, D), wid * NBLK + blk, pl.ds(0, 128)],
                osem[cur],
            )

        row_sel = [lax.iota(jnp.int32, 16) + 16 * t for t in range(8)]

        def transpose(cur):
            @pl.loop(0, D)
            def _(d):
                col_sel = jnp.full((16,), 0, jnp.int32) + d
                for t in range(8):
                    v = plsc.load_gather(rows_v.at[cur], [row_sel[t], col_sel])
                    rowsT_v[cur, d, pl.ds(16 * t, 16)] = v

        def do_slab(w, blk, first, start_next):
            cur = blk % 2
            gather_desc(w, blk, cur).wait()
            if not first:
                out_desc(w, blk, cur).wait()
            transpose(cur)
            if start_next is not None:
                gather_desc(start_next[0], start_next[1], cur).start()
            out_desc(w, blk, cur).start()

        gather_desc(0, 0, 0).start()
        gather_desc(0, 1, 1).start()

        do_slab(0, 0, True, (0, 2))
        do_slab(0, 1, True, (0, 3))
        do_slab(0, 2, False, (1, 0))
        do_slab(0, 3, False, (1, 1))

        @pl.loop(1, SEQ - 1)
        def _(w):
            do_slab(w, 0, False, (w, 2))
            do_slab(w, 1, False, (w, 3))
            do_slab(w, 2, False, (w + 1, 0))
            do_slab(w, 3, False, (w + 1, 1))

        w_last = SEQ - 1
        do_slab(w_last, 0, False, (w_last, 2))
        do_slab(w_last, 1, False, (w_last, 3))
        do_slab(w_last, 2, False, None)
        do_slab(w_last, 3, False, None)
        out_desc(w_last, 2, 0).wait()
        out_desc(w_last, 3, 1).wait()

    return kern(idx2, table2)


def kernel(x, weight):
    bsz, seq = x.shape
    n_vocab = weight.shape[0]
    idx2 = x.T.astype(jnp.int32) * 2
    table2 = jnp.pad(weight, ((0, 0), (0, 128 - D))).reshape(2 * n_vocab, D)
    out4 = _gather(idx2, table2)
    return out4.transpose(2, 3, 0, 1).reshape(bsz, seq, D)


# speedup vs baseline: 2.2259x; 2.2259x over previous
import functools

import jax
import jax.numpy as jnp
from jax import lax
from jax.experimental import pallas as pl
from jax.experimental.pallas import tpu as pltpu
from jax.experimental.pallas import tpu_sc as plsc

NC, NS = 2, 16
NW = NC * NS
D = 64
DP = 128
CHUNK = 256
NBUF = 4


@functools.partial(jax.jit, static_argnames=("total",))
def _gather(idx2, table2, total):
    b_per_w = total // NW
    n_chunks = b_per_w // CHUNK
    n_waves = n_chunks // NBUF
    mesh = plsc.VectorSubcoreMesh(
        core_axis_name="c", subcore_axis_name="s", num_cores=NC, num_subcores=NS
    )

    @functools.partial(
        pl.kernel,
        mesh=mesh,
        out_type=jax.ShapeDtypeStruct((total, DP), jnp.float32),
        scratch_types=[
            pltpu.VMEM((b_per_w,), jnp.int32),
            pltpu.VMEM((NBUF, CHUNK, D), jnp.float32),
            pltpu.SemaphoreType.DMA,
            pltpu.SemaphoreType.DMA,
            pltpu.SemaphoreType.DMA,
            pltpu.SemaphoreType.DMA,
            pltpu.SemaphoreType.DMA,
            pltpu.SemaphoreType.DMA,
            pltpu.SemaphoreType.DMA,
            pltpu.SemaphoreType.DMA,
        ],
        compiler_params=pltpu.CompilerParams(use_tc_tiling_on_sc=False),
    )
    def kern(idx_hbm, table_hbm, out_hbm, idx_v, rows_v, g0, g1, g2, g3, o0, o1, o2, o3):
        gsem = (g0, g1, g2, g3)
        osem = (o0, o1, o2, o3)
        wid = lax.axis_index("s") * NC + lax.axis_index("c")
        base = wid * b_per_w
        pltpu.sync_copy(idx_hbm.at[pl.ds(base, b_per_w)], idx_v)

        def gather_desc(c, b):
            return pltpu.make_async_copy(
                table_hbm.at[idx_v.at[pl.ds(c * CHUNK, CHUNK)]],
                rows_v.at[b],
                gsem[b],
            )

        def out_desc(c, b):
            return pltpu.make_async_copy(
                rows_v.at[b],
                out_hbm.at[pl.ds(base + c * CHUNK, CHUNK), pl.ds(0, D)],
                osem[b],
            )

        for b in range(NBUF):
            gather_desc(b, b).start()

        @pl.loop(0, n_waves - 1)
        def _(p):
            c = p * NBUF
            for b in range(NBUF):
                gather_desc(c + b, b).wait()
                out_desc(c + b, b).start()
            for b in range(NBUF):
                out_desc(c + b, b).wait()
                gather_desc(c + NBUF + b, b).start()

        c_last = (n_waves - 1) * NBUF
        for b in range(NBUF):
            gather_desc(c_last + b, b).wait()
            out_desc(c_last + b, b).start()
        for b in range(NBUF):
            out_desc(c_last + b, b).wait()

    return kern(idx2, table2)


def kernel(x, weight):
    bsz, seq = x.shape
    n_vocab = weight.shape[0]
    seq_p = 56
    pad_idx = (
        jnp.arange(seq_p - seq, dtype=jnp.int32)[None, :]
        + jnp.arange(bsz, dtype=jnp.int32)[:, None] * 61
    ) % n_vocab
    xp = jnp.concatenate([x.astype(jnp.int32), pad_idx], axis=1)
    idx2 = xp.reshape(bsz * seq_p) * 2
    table2 = jnp.pad(weight, ((0, 0), (0, DP - D))).reshape(2 * n_vocab, D)
    out = _gather(idx2, table2, bsz * seq_p)
    return out.reshape(bsz, seq_p, DP)[:, :seq, :D]
